# Initial kernel scaffold; baseline (speedup 1.0000x reference)
import functools

import jax
import jax.numpy as jnp
from jax import lax
from jax.experimental import pallas as pl
from jax.experimental.pallas import tpu as pltpu
from jax.experimental.pallas import tpu_sc as plsc

N = 10000
E = 320000
D = 128
ET = E + N

NC = 2
NS = 16
NW = NC * NS
CHUNK = 128
CPW = -(-ET // (NW * CHUNK))
TOT = NW * CPW * CHUNK

R = 10240
PT = R // NW * NC
DEGW = 16


def _mesh():
    return plsc.VectorSubcoreMesh(
        core_axis_name="c", subcore_axis_name="s", num_cores=NC,
        num_subcores=NS)


def _make_deg_kernel():
    @functools.partial(
        pl.kernel,
        out_type=jax.ShapeDtypeStruct((NC, R, DEGW), jnp.float32),
        mesh=_mesh(),
        scratch_types=[
            pltpu.VMEM((CHUNK, DEGW), jnp.float32),
            pltpu.VMEM((CHUNK, DEGW), jnp.float32),
            pltpu.VMEM((2, CHUNK), jnp.int32),
            pltpu.VMEM_SHARED((R, DEGW), jnp.float32),
        ],
    )
    def deg_kernel(dst_hbm, out_hbm, ones_v, zeros_v, idx_v, acc):
        cid = lax.axis_index("c")
        sid = lax.axis_index("s")
        wid = cid * NS + sid

        def init_body(i, _):
            ones_v[i, :] = jnp.full((DEGW,), 1.0, jnp.float32)
            zeros_v[i, :] = jnp.zeros((DEGW,), jnp.float32)
            return 0
        lax.fori_loop(0, CHUNK, init_body, 0)

        row0 = sid * PT
        for k in range(PT // CHUNK):
            pltpu.sync_copy(zeros_v, acc.at[pl.ds(row0 + k * CHUNK, CHUNK)])
        plsc.subcore_barrier()

        def body(j, _):
            base = pl.multiple_of((wid * CPW + j) * CHUNK, CHUNK)
            pltpu.sync_copy(dst_hbm.at[pl.ds(base, CHUNK)], idx_v.at[0])
            pltpu.sync_copy(ones_v, acc.at[idx_v.at[0]], add=True)
            return 0
        lax.fori_loop(0, CPW, body, 0)

        plsc.subcore_barrier()
        pltpu.sync_copy(acc.at[pl.ds(row0, PT)],
                        out_hbm.at[cid, pl.ds(row0, PT)])

    return deg_kernel


def _make_agg_kernel():
    @functools.partial(
        pl.kernel,
        out_type=jax.ShapeDtypeStruct((NC, R, D), jnp.float32),
        mesh=_mesh(),
        scratch_types=[
            pltpu.VMEM((CHUNK, D), jnp.float32),
            pltpu.VMEM((CHUNK, D), jnp.float32),
            pltpu.VMEM((2, CHUNK), jnp.int32),
            pltpu.VMEM((2, CHUNK), jnp.int32),
            pltpu.VMEM_SHARED((R, D), jnp.float32),
            pltpu.SemaphoreType.DMA,
        ],
    )
    def agg_kernel(g_hbm, src_hbm, dst_hbm, out_hbm,
                   rows_v, zeros_v, sidx_v, didx_v, acc, gsem):
        cid = lax.axis_index("c")
        sid = lax.axis_index("s")
        wid = cid * NS + sid

        def zbody(i, _):
            for c in range(D // 16):
                zeros_v[i, pl.ds(c * 16, 16)] = jnp.zeros((16,), jnp.float32)
            return 0
        lax.fori_loop(0, CHUNK, zbody, 0)

        row0 = sid * PT
        for k in range(PT // CHUNK):
            pltpu.sync_copy(zeros_v, acc.at[pl.ds(row0 + k * CHUNK, CHUNK)])
        plsc.subcore_barrier()

        def body(j, _):
            base = pl.multiple_of((wid * CPW + j) * CHUNK, CHUNK)
            pltpu.sync_copy(src_hbm.at[pl.ds(base, CHUNK)], sidx_v.at[0])
            pltpu.sync_copy(dst_hbm.at[pl.ds(base, CHUNK)], didx_v.at[0])
            pltpu.async_copy(g_hbm.at[sidx_v.at[0]], rows_v, gsem).wait()
            pltpu.sync_copy(rows_v, acc.at[didx_v.at[0]], add=True)
            return 0
        lax.fori_loop(0, CPW, body, 0)

        plsc.subcore_barrier()
        pltpu.sync_copy(acc.at[pl.ds(row0, PT)],
                        out_hbm.at[cid, pl.ds(row0, PT)])

    return agg_kernel


def _dinv_block(degp):
    deg = degp[0, :, 0:1] + degp[1, :, 0:1]
    return jnp.where(deg > 0.0, lax.rsqrt(jnp.maximum(deg, 1e-12)), 0.0)


def _tc1_body(degp_ref, x_ref, w_ref, g_ref):
    dinv = _dinv_block(degp_ref[...])
    h = jnp.dot(x_ref[...], w_ref[...], preferred_element_type=jnp.float32)
    g_ref[...] = h * dinv


def _tc2_body(aggp_ref, degp_ref, b1_ref, w2_ref, g_ref):
    dinv = _dinv_block(degp_ref[...])
    s = aggp_ref[0] + aggp_ref[1]
    h1 = jnp.maximum(s * dinv + b1_ref[...], 0.0)
    g_ref[...] = jnp.dot(h1, w2_ref[...],
                         preferred_element_type=jnp.float32) * dinv


def _tc3_body(aggp_ref, degp_ref, b2_ref, out_ref):
    dinv = _dinv_block(degp_ref[...])
    out_ref[...] = (aggp_ref[0] + aggp_ref[1]) * dinv + b2_ref[...]


_TB = 1024


def _degp_spec():
    return pl.BlockSpec((NC, _TB, DEGW), lambda i: (0, i, 0))


def _aggp_spec():
    return pl.BlockSpec((NC, _TB, D), lambda i: (0, i, 0))


def _row_spec():
    return pl.BlockSpec((_TB, D), lambda i: (i, 0))


def _full_spec():
    return pl.BlockSpec((D, D), lambda i: (0, 0))


def _bias_spec():
    return pl.BlockSpec((1, D), lambda i: (0, 0))


def _tc1(degp, x_pad, W1):
    return pl.pallas_call(
        _tc1_body,
        out_shape=jax.ShapeDtypeStruct((R, D), jnp.float32),
        grid=(R // _TB,),
        in_specs=[_degp_spec(), _row_spec(), _full_spec()],
        out_specs=_row_spec(),
    )(degp, x_pad, W1)


def _tc2(aggp, degp, b1, W2):
    return pl.pallas_call(
        _tc2_body,
        out_shape=jax.ShapeDtypeStruct((R, D), jnp.float32),
        grid=(R // _TB,),
        in_specs=[_aggp_spec(), _degp_spec(), _bias_spec(), _full_spec()],
        out_specs=_row_spec(),
    )(aggp, degp, b1, W2)


def _tc3(aggp, degp, b2):
    return pl.pallas_call(
        _tc3_body,
        out_shape=jax.ShapeDtypeStruct((R, D), jnp.float32),
        grid=(R // _TB,),
        in_specs=[_aggp_spec(), _degp_spec(), _bias_spec()],
        out_specs=_row_spec(),
    )(aggp, degp, b2)


def kernel(x, edge_index, W1, b1, W2, b2):
    loop = jnp.arange(N, dtype=jnp.int32)
    src = jnp.concatenate([edge_index[0].astype(jnp.int32), loop])
    dst = jnp.concatenate([edge_index[1].astype(jnp.int32), loop])
    src = jnp.pad(src, (0, TOT - ET))
    dst = jnp.pad(dst, (0, TOT - ET), constant_values=N)
    x_pad = jnp.pad(x, ((0, R - N), (0, 0)))

    degp = _make_deg_kernel()(dst)
    g1 = _tc1(degp, x_pad, W1)
    aggp1 = _make_agg_kernel()(g1, src, dst)
    g2 = _tc2(aggp1, degp, b1.reshape(1, D), W2)
    aggp2 = _make_agg_kernel()(g2, src, dst)
    out = _tc3(aggp2, degp, b2.reshape(1, D))
    return out[:N]


# speedup vs baseline: 12.6356x; 12.6356x over previous
import functools

import jax
import jax.numpy as jnp
from jax import lax
from jax.experimental import pallas as pl
from jax.experimental.pallas import tpu as pltpu
from jax.experimental.pallas import tpu_sc as plsc

N = 10000
E = 320000
D = 128
ET = E + N

NC = 2
NS = 16
NW = NC * NS
CHUNK = 128
CPW = -(-ET // (NW * CHUNK))
TOT = NW * CPW * CHUNK

R = 10240
PT = R // NW * NC
DEGW = 128


def _mesh():
    return plsc.VectorSubcoreMesh(
        core_axis_name="c", subcore_axis_name="s", num_cores=NC,
        num_subcores=NS)


def _make_deg_kernel():
    @functools.partial(
        pl.kernel,
        out_type=jax.ShapeDtypeStruct((NC, R, DEGW), jnp.float32),
        mesh=_mesh(),
        scratch_types=[
            pltpu.VMEM((CHUNK, DEGW), jnp.float32),
            pltpu.VMEM((CHUNK, DEGW), jnp.float32),
            pltpu.VMEM((2, CHUNK), jnp.int32),
            pltpu.VMEM_SHARED((R, DEGW), jnp.float32),
        ],
    )
    def deg_kernel(dst_hbm, out_hbm, ones_v, zeros_v, idx_v, acc):
        cid = lax.axis_index("c")
        sid = lax.axis_index("s")
        wid = cid * NS + sid

        def init_body(i, _):
            for c in range(DEGW // 16):
                ones_v[i, pl.ds(c * 16, 16)] = jnp.full((16,), 1.0,
                                                        jnp.float32)
                zeros_v[i, pl.ds(c * 16, 16)] = jnp.zeros((16,), jnp.float32)
            return 0
        lax.fori_loop(0, CHUNK, init_body, 0)

        row0 = sid * PT
        for k in range(PT // CHUNK):
            pltpu.sync_copy(zeros_v, acc.at[pl.ds(row0 + k * CHUNK, CHUNK)])
        plsc.subcore_barrier()

        def body(j, _):
            base = pl.multiple_of((wid * CPW + j) * CHUNK, CHUNK)
            pltpu.sync_copy(dst_hbm.at[pl.ds(base, CHUNK)], idx_v.at[0])
            pltpu.sync_copy(ones_v, acc.at[idx_v.at[0]], add=True)
            return 0
        lax.fori_loop(0, CPW, body, 0)

        plsc.subcore_barrier()
        pltpu.sync_copy(acc.at[pl.ds(row0, PT)],
                        out_hbm.at[cid, pl.ds(row0, PT)])

    return deg_kernel


def _make_agg_kernel():
    @functools.partial(
        pl.kernel,
        out_type=jax.ShapeDtypeStruct((NC, R, D), jnp.float32),
        mesh=_mesh(),
        scratch_types=[
            pltpu.VMEM((CHUNK, D), jnp.float32),
            pltpu.VMEM((CHUNK, D), jnp.float32),
            pltpu.VMEM((2, CHUNK), jnp.int32),
            pltpu.VMEM((2, CHUNK), jnp.int32),
            pltpu.VMEM_SHARED((R, D), jnp.float32),
            pltpu.SemaphoreType.DMA,
        ],
    )
    def agg_kernel(g_hbm, src_hbm, dst_hbm, out_hbm,
                   rows_v, zeros_v, sidx_v, didx_v, acc, gsem):
        cid = lax.axis_index("c")
        sid = lax.axis_index("s")
        wid = cid * NS + sid

        def zbody(i, _):
            for c in range(D // 16):
                zeros_v[i, pl.ds(c * 16, 16)] = jnp.zeros((16,), jnp.float32)
            return 0
        lax.fori_loop(0, CHUNK, zbody, 0)

        row0 = sid * PT
        for k in range(PT // CHUNK):
            pltpu.sync_copy(zeros_v, acc.at[pl.ds(row0 + k * CHUNK, CHUNK)])
        plsc.subcore_barrier()

        def body(j, _):
            base = pl.multiple_of((wid * CPW + j) * CHUNK, CHUNK)
            pltpu.sync_copy(src_hbm.at[pl.ds(base, CHUNK)], sidx_v.at[0])
            pltpu.sync_copy(dst_hbm.at[pl.ds(base, CHUNK)], didx_v.at[0])
            pltpu.async_copy(g_hbm.at[sidx_v.at[0]], rows_v, gsem).wait()
            pltpu.sync_copy(rows_v, acc.at[didx_v.at[0]], add=True)
            return 0
        lax.fori_loop(0, CPW, body, 0)

        plsc.subcore_barrier()
        pltpu.sync_copy(acc.at[pl.ds(row0, PT)],
                        out_hbm.at[cid, pl.ds(row0, PT)])

    return agg_kernel


def _dinv_block(degp):
    deg = degp[0, :, 0:1] + degp[1, :, 0:1]
    return jnp.where(deg > 0.0, lax.rsqrt(jnp.maximum(deg, 1e-12)), 0.0)


def _tc1_body(degp_ref, x_ref, w_ref, g_ref):
    dinv = _dinv_block(degp_ref[...])
    h = jnp.dot(x_ref[...], w_ref[...], preferred_element_type=jnp.float32)
    g_ref[...] = h * dinv


def _tc2_body(aggp_ref, degp_ref, b1_ref, w2_ref, g_ref):
    dinv = _dinv_block(degp_ref[...])
    s = aggp_ref[0] + aggp_ref[1]
    h1 = jnp.maximum(s * dinv + b1_ref[...], 0.0)
    g_ref[...] = jnp.dot(h1, w2_ref[...],
                         preferred_element_type=jnp.float32) * dinv


def _tc3_body(aggp_ref, degp_ref, b2_ref, out_ref):
    dinv = _dinv_block(degp_ref[...])
    out_ref[...] = (aggp_ref[0] + aggp_ref[1]) * dinv + b2_ref[...]


_TB = 1024


def _degp_spec():
    return pl.BlockSpec((NC, _TB, DEGW), lambda i: (0, i, 0))


def _aggp_spec():
    return pl.BlockSpec((NC, _TB, D), lambda i: (0, i, 0))


def _row_spec():
    return pl.BlockSpec((_TB, D), lambda i: (i, 0))


def _full_spec():
    return pl.BlockSpec((D, D), lambda i: (0, 0))


def _bias_spec():
    return pl.BlockSpec((1, D), lambda i: (0, 0))


def _tc1(degp, x_pad, W1):
    return pl.pallas_call(
        _tc1_body,
        out_shape=jax.ShapeDtypeStruct((R, D), jnp.float32),
        grid=(R // _TB,),
        in_specs=[_degp_spec(), _row_spec(), _full_spec()],
        out_specs=_row_spec(),
    )(degp, x_pad, W1)


def _tc2(aggp, degp, b1, W2):
    return pl.pallas_call(
        _tc2_body,
        out_shape=jax.ShapeDtypeStruct((R, D), jnp.float32),
        grid=(R // _TB,),
        in_specs=[_aggp_spec(), _degp_spec(), _bias_spec(), _full_spec()],
        out_specs=_row_spec(),
    )(aggp, degp, b1, W2)


def _tc3(aggp, degp, b2):
    return pl.pallas_call(
        _tc3_body,
        out_shape=jax.ShapeDtypeStruct((R, D), jnp.float32),
        grid=(R // _TB,),
        in_specs=[_aggp_spec(), _degp_spec(), _bias_spec()],
        out_specs=_row_spec(),
    )(aggp, degp, b2)


def kernel(x, edge_index, W1, b1, W2, b2):
    loop = jnp.arange(N, dtype=jnp.int32)
    src = jnp.concatenate([edge_index[0].astype(jnp.int32), loop])
    dst = jnp.concatenate([edge_index[1].astype(jnp.int32), loop])
    src = jnp.pad(src, (0, TOT - ET))
    dst = jnp.pad(dst, (0, TOT - ET), constant_values=N)
    x_pad = jnp.pad(x, ((0, R - N), (0, 0)))

    degp = _make_deg_kernel()(dst)
    g1 = _tc1(degp, x_pad, W1)
    aggp1 = _make_agg_kernel()(g1, src, dst)
    g2 = _tc2(aggp1, degp, b1.reshape(1, D), W2)
    aggp2 = _make_agg_kernel()(g2, src, dst)
    out = _tc3(aggp2, degp, b2.reshape(1, D))
    return out[:N]
